# Initial kernel scaffold; baseline (speedup 1.0000x reference)
import jax
import jax.numpy as jnp
from jax.experimental import pallas as pl


def kernel(node_states, relations, W1, b1, W2, b2, U1, u1, U2, u2):
    raise NotImplementedError("write your pallas kernel here")


# speedup vs baseline: 2.2811x; 2.2811x over previous
import functools
import jax
import jax.numpy as jnp
from jax import lax
from jax.experimental import pallas as pl
from jax.experimental.pallas import tpu as pltpu
from jax.experimental.pallas import tpu_sc as plsc

NC = 2
NS = 16
NW = NC * NS
CH = 128
KQ = 4
WQ = 64


def _tc_tables(node_states, Wcat, bcat):
    n, h = node_states.shape
    bn = 1000

    def body(ns_ref, w_ref, b_ref, *outs):
        pq = jnp.dot(ns_ref[...], w_ref[...],
                     preferred_element_type=jnp.float32) + b_ref[...]
        for q in range(2 * KQ):
            outs[q][...] = pq[:, WQ * q:WQ * (q + 1)]

    return pl.pallas_call(
        body,
        grid=(n // bn,),
        in_specs=[
            pl.BlockSpec((bn, h), lambda i: (i, 0)),
            pl.BlockSpec((h, 4 * h), lambda i: (0, 0)),
            pl.BlockSpec((1, 4 * h), lambda i: (0, 0)),
        ],
        out_specs=[pl.BlockSpec((bn, WQ), lambda i: (i, 0))
                   for _ in range(2 * KQ)],
        out_shape=[jax.ShapeDtypeStruct((n, WQ), jnp.float32)
                   for _ in range(2 * KQ)],
    )(node_states, Wcat, bcat)


def _sc_scatter(ptabs, qtabs, i0, i1, z64, z16, one16, n):
    ntup = i0.shape[0]
    nchunks = ntup // CH
    per_w = nchunks // NW
    rem = nchunks - per_w * NW
    rows = (n // NS) // 8 * 8
    tail = n - rows * NS

    mesh = plsc.VectorSubcoreMesh(core_axis_name="c", subcore_axis_name="s",
                                  num_cores=NC, num_subcores=NS)

    @functools.partial(
        pl.kernel,
        out_type=[
            jax.ShapeDtypeStruct((KQ, NC, n, WQ), jnp.float32),
            jax.ShapeDtypeStruct((KQ, NC, n, WQ), jnp.float32),
            jax.ShapeDtypeStruct((NC, n, 16), jnp.float32),
            jax.ShapeDtypeStruct((NC, n, 16), jnp.float32),
        ],
        mesh=mesh,
        scratch_types=[
            pltpu.VMEM_SHARED((n, WQ), jnp.float32),
            pltpu.VMEM_SHARED((n, WQ), jnp.float32),
            pltpu.VMEM_SHARED((n, 16), jnp.float32),
            pltpu.VMEM_SHARED((n, 16), jnp.float32),
            pltpu.VMEM((1, CH), jnp.int32),
            pltpu.VMEM((1, CH), jnp.int32),
            pltpu.VMEM((CH, WQ), jnp.float32),
            pltpu.VMEM((CH, WQ), jnp.float32),
            pltpu.VMEM((CH, 16), jnp.float32),
            pltpu.SemaphoreType.DMA,
            pltpu.SemaphoreType.DMA,
        ],
        compiler_params=pltpu.CompilerParams(use_tc_tiling_on_sc=False),
    )
    def k(p0, p1, p2, p3, q0, q1, q2, q3, i0_h, i1_h, z64_h, z16_h, one16_h,
          s0_out, s1_out, c0_out, c1_out,
          s0_sh, s1_sh, c0_sh, c1_sh, idx0, idx1, bufp, bufq, ones,
          semp, semq):
        ptab = (p0, p1, p2, p3)
        qtab = (q0, q1, q2, q3)
        cid = lax.axis_index("c")
        sid = lax.axis_index("s")
        wid = sid * NC + cid
        nj = per_w + (wid < rem).astype(jnp.int32)
        pltpu.sync_copy(one16_h, ones)

        for kq in range(KQ):
            sl = pl.ds(sid * rows, rows)
            tl = pl.ds(NS * rows, tail)
            zt64 = z64_h.at[pl.ds(0, tail)]
            zt16 = z16_h.at[pl.ds(0, tail)]
            pltpu.sync_copy(z64_h, s0_sh.at[sl])
            pltpu.sync_copy(z64_h, s1_sh.at[sl])
            if kq == 0:
                pltpu.sync_copy(z16_h, c0_sh.at[sl])
                pltpu.sync_copy(z16_h, c1_sh.at[sl])

            @pl.when(sid == NS - 1)
            def _():
                pltpu.sync_copy(zt64, s0_sh.at[tl])
                pltpu.sync_copy(zt64, s1_sh.at[tl])
                if kq == 0:
                    pltpu.sync_copy(zt16, c0_sh.at[tl])
                    pltpu.sync_copy(zt16, c1_sh.at[tl])

            plsc.subcore_barrier()

            def chunk(j, carry):
                base = (wid + NW * j) * CH
                pltpu.sync_copy(i0_h.at[pl.ds(base, CH)], idx0.at[0])
                pltpu.sync_copy(i1_h.at[pl.ds(base, CH)], idx1.at[0])
                cp = pltpu.async_copy(ptab[kq].at[idx0.at[0]], bufp, semp)
                cq = pltpu.async_copy(qtab[kq].at[idx1.at[0]], bufq, semq)
                cp.wait()
                cq.wait()

                def row(r, c):
                    for q in range(WQ // 16):
                        sl = pl.ds(q * 16, 16)
                        bufp[r, sl] = jnp.maximum(bufp[r, sl] + bufq[r, sl],
                                                  0.0)
                    return c

                lax.fori_loop(0, CH, row, 0)
                pltpu.sync_copy(bufp, s0_sh.at[idx0.at[0]], add=True)
                pltpu.sync_copy(bufp, s1_sh.at[idx1.at[0]], add=True)
                if kq == 0:
                    pltpu.sync_copy(ones, c0_sh.at[idx0.at[0]], add=True)
                    pltpu.sync_copy(ones, c1_sh.at[idx1.at[0]], add=True)
                return carry

            lax.fori_loop(0, nj, chunk, 0)
            plsc.subcore_barrier()

            pltpu.sync_copy(s0_sh.at[sl], s0_out.at[kq, cid, sl])
            pltpu.sync_copy(s1_sh.at[sl], s1_out.at[kq, cid, sl])
            if kq == 0:
                pltpu.sync_copy(c0_sh.at[sl], c0_out.at[cid, sl])
                pltpu.sync_copy(c1_sh.at[sl], c1_out.at[cid, sl])

            @pl.when(sid == NS - 1)
            def _():
                pltpu.sync_copy(s0_sh.at[tl], s0_out.at[kq, cid, tl])
                pltpu.sync_copy(s1_sh.at[tl], s1_out.at[kq, cid, tl])
                if kq == 0:
                    pltpu.sync_copy(c0_sh.at[tl], c0_out.at[cid, tl])
                    pltpu.sync_copy(c1_sh.at[tl], c1_out.at[cid, tl])

            plsc.subcore_barrier()

    return k(*ptabs, *qtabs, i0, i1, z64, z16, one16)


def _tc_update(s0p, s1p, c0p, c1p, node_states, W2ab, b2a, b2b,
               U1a, U1b, u1r, U2t, u2r):
    n, h = node_states.shape
    bn = 1000

    def body(s0_ref, s1_ref, c0_ref, c1_ref, ns_ref, w2_ref, b2a_ref, b2b_ref,
             u1a_ref, u1b_ref, u1_ref, u2t_ref, u2_ref, out_ref):
        s0 = jnp.concatenate(
            [s0_ref[kq, 0] + s0_ref[kq, 1] for kq in range(KQ)], axis=1)
        s1 = jnp.concatenate(
            [s1_ref[kq, 0] + s1_ref[kq, 1] for kq in range(KQ)], axis=1)
        c0 = c0_ref[0, :, 0] + c0_ref[1, :, 0]
        c1 = c1_ref[0, :, 0] + c1_ref[1, :, 0]
        s01 = jnp.concatenate([s0, s1], axis=1)
        summ = jnp.dot(s01, w2_ref[...], preferred_element_type=jnp.float32)
        summ = summ + c0[:, None] * b2a_ref[...] + c1[:, None] * b2b_ref[...]
        z = jnp.maximum(
            jnp.dot(summ, u1a_ref[...], preferred_element_type=jnp.float32)
            + jnp.dot(ns_ref[...], u1b_ref[...],
                      preferred_element_type=jnp.float32)
            + u1_ref[...], 0.0)
        out_ref[...] = (jnp.dot(z, u2t_ref[...],
                                preferred_element_type=jnp.float32)
                        + u2_ref[...])

    return pl.pallas_call(
        body,
        grid=(n // bn,),
        in_specs=[
            pl.BlockSpec((KQ, NC, bn, WQ), lambda i: (0, 0, i, 0)),
            pl.BlockSpec((KQ, NC, bn, WQ), lambda i: (0, 0, i, 0)),
            pl.BlockSpec((NC, bn, 16), lambda i: (0, i, 0)),
            pl.BlockSpec((NC, bn, 16), lambda i: (0, i, 0)),
            pl.BlockSpec((bn, h), lambda i: (i, 0)),
            pl.BlockSpec((4 * h, h), lambda i: (0, 0)),
            pl.BlockSpec((1, h), lambda i: (0, 0)),
            pl.BlockSpec((1, h), lambda i: (0, 0)),
            pl.BlockSpec((h, 2 * h), lambda i: (0, 0)),
            pl.BlockSpec((h, 2 * h), lambda i: (0, 0)),
            pl.BlockSpec((1, 2 * h), lambda i: (0, 0)),
            pl.BlockSpec((2 * h, h), lambda i: (0, 0)),
            pl.BlockSpec((1, h), lambda i: (0, 0)),
        ],
        out_specs=pl.BlockSpec((bn, h), lambda i: (i, 0)),
        out_shape=jax.ShapeDtypeStruct((n, h), jnp.float32),
    )(s0p, s1p, c0p, c1p, node_states, W2ab, b2a, b2b,
      U1a, U1b, u1r, U2t, u2r)


def kernel(node_states, relations, W1, b1, W2, b2, U1, u1, U2, u2):
    n, h = node_states.shape
    pairs = relations.reshape(-1, 2)
    i0 = pairs[:, 0]
    i1 = pairs[:, 1]

    A = W1[:, :h].T
    B = W1[:, h:].T
    Wcat = jnp.concatenate([A, B], axis=1)
    bcat = jnp.concatenate([b1, jnp.zeros_like(b1)]).reshape(1, 4 * h)
    W2ab = jnp.concatenate([W2[:h].T, W2[h:].T], axis=0)
    b2a = b2[:h].reshape(1, h)
    b2b = b2[h:].reshape(1, h)
    U1a = U1[:, :h].T
    U1b = U1[:, h:].T
    u1r = u1.reshape(1, 2 * h)
    U2t = U2.T
    u2r = u2.reshape(1, h)

    z64 = jnp.zeros(((n // NS) // 8 * 8, WQ), jnp.float32)
    z16 = jnp.zeros(((n // NS) // 8 * 8, 16), jnp.float32)
    one16 = jnp.ones((CH, 16), jnp.float32)

    tabs = _tc_tables(node_states, Wcat, bcat)
    ptabs, qtabs = tabs[:KQ], tabs[KQ:]
    s0p, s1p, c0p, c1p = _sc_scatter(ptabs, qtabs, i0, i1, z64, z16, one16, n)
    return _tc_update(s0p, s1p, c0p, c1p, node_states, W2ab, b2a, b2b,
                      U1a, U1b, u1r, U2t, u2r)


# speedup vs baseline: 3.4390x; 1.5076x over previous
import functools
import jax
import jax.numpy as jnp
from jax import lax
from jax.experimental import pallas as pl
from jax.experimental.pallas import tpu as pltpu
from jax.experimental.pallas import tpu_sc as plsc

NC = 2
NS = 16
NW = NC * NS
CH = 128
KQ = 4
WQ = 64


def _tc_tables(node_states, Wcat, bcat):
    n, h = node_states.shape
    bn = 1000

    def body(ns_ref, w_ref, b_ref, *outs):
        pq = jnp.dot(ns_ref[...], w_ref[...],
                     preferred_element_type=jnp.float32) + b_ref[...]
        for q in range(2 * KQ):
            outs[q][...] = pq[:, WQ * q:WQ * (q + 1)]

    return pl.pallas_call(
        body,
        grid=(n // bn,),
        in_specs=[
            pl.BlockSpec((bn, h), lambda i: (i, 0)),
            pl.BlockSpec((h, 4 * h), lambda i: (0, 0)),
            pl.BlockSpec((1, 4 * h), lambda i: (0, 0)),
        ],
        out_specs=[pl.BlockSpec((bn, WQ), lambda i: (i, 0))
                   for _ in range(2 * KQ)],
        out_shape=[jax.ShapeDtypeStruct((n, WQ), jnp.float32)
                   for _ in range(2 * KQ)],
    )(node_states, Wcat, bcat)


def _sc_counts(i0r, i1r, z16, one16, n, nchunks):
    per_w = nchunks // NW
    rem = nchunks - per_w * NW
    njmax = per_w + (1 if rem else 0)
    rows = (n // NS) // 8 * 8
    tail = n - rows * NS

    mesh = plsc.VectorSubcoreMesh(core_axis_name="c", subcore_axis_name="s",
                                  num_cores=NC, num_subcores=NS)

    @functools.partial(
        pl.kernel,
        out_type=[
            jax.ShapeDtypeStruct((NC, n, 16), jnp.float32),
            jax.ShapeDtypeStruct((NC, n, 16), jnp.float32),
        ],
        mesh=mesh,
        scratch_types=[
            pltpu.VMEM_SHARED((n, 16), jnp.float32),
            pltpu.VMEM_SHARED((n, 16), jnp.float32),
            pltpu.VMEM((njmax, CH), jnp.int32),
            pltpu.VMEM((njmax, CH), jnp.int32),
            pltpu.VMEM((CH, 16), jnp.float32),
        ],
        compiler_params=pltpu.CompilerParams(use_tc_tiling_on_sc=False),
    )
    def k(i0_h, i1_h, z16_h, one16_h, c0_out, c1_out,
          c0_sh, c1_sh, idx0, idx1, ones):
        cid = lax.axis_index("c")
        sid = lax.axis_index("s")
        wid = sid * NC + cid
        nj = per_w + (wid < rem).astype(jnp.int32)
        row0 = wid * per_w + jnp.minimum(wid, rem)
        pltpu.sync_copy(one16_h, ones)
        pltpu.sync_copy(i0_h.at[pl.ds(row0, njmax)], idx0)
        pltpu.sync_copy(i1_h.at[pl.ds(row0, njmax)], idx1)
        sl = pl.ds(sid * rows, rows)
        tl = pl.ds(NS * rows, tail)
        pltpu.sync_copy(z16_h, c0_sh.at[sl])
        pltpu.sync_copy(z16_h, c1_sh.at[sl])

        @pl.when(sid == NS - 1)
        def _():
            pltpu.sync_copy(z16_h.at[pl.ds(0, tail)], c0_sh.at[tl])
            pltpu.sync_copy(z16_h.at[pl.ds(0, tail)], c1_sh.at[tl])

        plsc.subcore_barrier()

        def chunk(j, carry):
            pltpu.sync_copy(ones, c0_sh.at[idx0.at[j]], add=True)
            pltpu.sync_copy(ones, c1_sh.at[idx1.at[j]], add=True)
            return carry

        lax.fori_loop(0, nj, chunk, 0)
        plsc.subcore_barrier()
        pltpu.sync_copy(c0_sh.at[sl], c0_out.at[cid, sl])
        pltpu.sync_copy(c1_sh.at[sl], c1_out.at[cid, sl])

        @pl.when(sid == NS - 1)
        def _():
            pltpu.sync_copy(c0_sh.at[tl], c0_out.at[cid, tl])
            pltpu.sync_copy(c1_sh.at[tl], c1_out.at[cid, tl])

        plsc.subcore_barrier()

    return k(i0r, i1r, z16, one16)


def _sc_scatter(ptabs, qtabs, i0r, i1r, z64, n, nchunks):
    per_w = nchunks // NW
    rem = nchunks - per_w * NW
    njmax = per_w + (1 if rem else 0)
    rows = (n // NS) // 8 * 8
    tail = n - rows * NS

    mesh = plsc.VectorSubcoreMesh(core_axis_name="c", subcore_axis_name="s",
                                  num_cores=NC, num_subcores=NS)

    @functools.partial(
        pl.kernel,
        out_type=[
            jax.ShapeDtypeStruct((KQ, NC, n, WQ), jnp.float32),
            jax.ShapeDtypeStruct((KQ, NC, n, WQ), jnp.float32),
        ],
        mesh=mesh,
        scratch_types=[
            pltpu.VMEM_SHARED((n, WQ), jnp.float32),
            pltpu.VMEM_SHARED((n, WQ), jnp.float32),
            pltpu.VMEM((njmax, CH), jnp.int32),
            pltpu.VMEM((njmax, CH), jnp.int32),
            pltpu.VMEM((2, CH, WQ), jnp.float32),
            pltpu.VMEM((2, CH, WQ), jnp.float32),
            pltpu.SemaphoreType.DMA,
            pltpu.SemaphoreType.DMA,
            pltpu.SemaphoreType.DMA,
            pltpu.SemaphoreType.DMA,
        ],
        compiler_params=pltpu.CompilerParams(use_tc_tiling_on_sc=False),
    )
    def k(p0, p1, p2, p3, q0, q1, q2, q3, i0_h, i1_h, z64_h,
          s0_out, s1_out,
          s0_sh, s1_sh, idx0, idx1, bufp, bufq,
          semp0, semp1, semq0, semq1):
        ptab = (p0, p1, p2, p3)
        qtab = (q0, q1, q2, q3)
        semps = (semp0, semp1)
        semqs = (semq0, semq1)
        cid = lax.axis_index("c")
        sid = lax.axis_index("s")
        wid = sid * NC + cid
        nj = per_w + (wid < rem).astype(jnp.int32)
        row0 = wid * per_w + jnp.minimum(wid, rem)
        pltpu.sync_copy(i0_h.at[pl.ds(row0, njmax)], idx0)
        pltpu.sync_copy(i1_h.at[pl.ds(row0, njmax)], idx1)

        for kq in range(KQ):
            sl = pl.ds(sid * rows, rows)
            tl = pl.ds(NS * rows, tail)
            zt64 = z64_h.at[pl.ds(0, tail)]
            pltpu.sync_copy(z64_h, s0_sh.at[sl])
            pltpu.sync_copy(z64_h, s1_sh.at[sl])

            @pl.when(sid == NS - 1)
            def _():
                pltpu.sync_copy(zt64, s0_sh.at[tl])
                pltpu.sync_copy(zt64, s1_sh.at[tl])

            plsc.subcore_barrier()

            def issue(j, b):
                pltpu.async_copy(ptab[kq].at[idx0.at[j]], bufp.at[b],
                                 semps[b])
                pltpu.async_copy(qtab[kq].at[idx1.at[j]], bufq.at[b],
                                 semqs[b])

            def process(j, b):
                pltpu.make_async_copy(ptab[kq].at[idx0.at[j]], bufp.at[b],
                                      semps[b]).wait()
                pltpu.make_async_copy(qtab[kq].at[idx1.at[j]], bufq.at[b],
                                      semqs[b]).wait()

                def row(r, c):
                    for q in range(WQ // 16):
                        sl = pl.ds(q * 16, 16)
                        bufp[b, r, sl] = jnp.maximum(
                            bufp[b, r, sl] + bufq[b, r, sl], 0.0)
                    return c

                lax.fori_loop(0, CH, row, 0)
                pltpu.sync_copy(bufp.at[b], s0_sh.at[idx0.at[j]], add=True)
                pltpu.sync_copy(bufp.at[b], s1_sh.at[idx1.at[j]], add=True)

            def pair(jj, carry):
                j0 = 2 * jj
                j1 = j0 + 1

                @pl.when(j1 < nj)
                def _():
                    issue(j1, 1)

                process(j0, 0)

                @pl.when(j1 + 1 < nj)
                def _():
                    issue(j1 + 1, 0)

                @pl.when(j1 < nj)
                def _():
                    process(j1, 1)

                return carry

            issue(0, 0)
            lax.fori_loop(0, (nj + 1) // 2, pair, 0)
            plsc.subcore_barrier()

            pltpu.sync_copy(s0_sh.at[sl], s0_out.at[kq, cid, sl])
            pltpu.sync_copy(s1_sh.at[sl], s1_out.at[kq, cid, sl])

            @pl.when(sid == NS - 1)
            def _():
                pltpu.sync_copy(s0_sh.at[tl], s0_out.at[kq, cid, tl])
                pltpu.sync_copy(s1_sh.at[tl], s1_out.at[kq, cid, tl])

            plsc.subcore_barrier()

    return k(*ptabs, *qtabs, i0r, i1r, z64)


def _tc_update(s0p, s1p, c0p, c1p, node_states, W2ab, b2a, b2b,
               U1a, U1b, u1r, U2t, u2r):
    n, h = node_states.shape
    bn = 1000

    def body(s0_ref, s1_ref, c0_ref, c1_ref, ns_ref, w2_ref, b2a_ref, b2b_ref,
             u1a_ref, u1b_ref, u1_ref, u2t_ref, u2_ref, out_ref):
        s0 = jnp.concatenate(
            [s0_ref[kq, 0] + s0_ref[kq, 1] for kq in range(KQ)], axis=1)
        s1 = jnp.concatenate(
            [s1_ref[kq, 0] + s1_ref[kq, 1] for kq in range(KQ)], axis=1)
        c0 = c0_ref[0, :, 0] + c0_ref[1, :, 0]
        c1 = c1_ref[0, :, 0] + c1_ref[1, :, 0]
        s01 = jnp.concatenate([s0, s1], axis=1)
        summ = jnp.dot(s01, w2_ref[...], preferred_element_type=jnp.float32)
        summ = summ + c0[:, None] * b2a_ref[...] + c1[:, None] * b2b_ref[...]
        z = jnp.maximum(
            jnp.dot(summ, u1a_ref[...], preferred_element_type=jnp.float32)
            + jnp.dot(ns_ref[...], u1b_ref[...],
                      preferred_element_type=jnp.float32)
            + u1_ref[...], 0.0)
        out_ref[...] = (jnp.dot(z, u2t_ref[...],
                                preferred_element_type=jnp.float32)
                        + u2_ref[...])

    return pl.pallas_call(
        body,
        grid=(n // bn,),
        in_specs=[
            pl.BlockSpec((KQ, NC, bn, WQ), lambda i: (0, 0, i, 0)),
            pl.BlockSpec((KQ, NC, bn, WQ), lambda i: (0, 0, i, 0)),
            pl.BlockSpec((NC, bn, 16), lambda i: (0, i, 0)),
            pl.BlockSpec((NC, bn, 16), lambda i: (0, i, 0)),
            pl.BlockSpec((bn, h), lambda i: (i, 0)),
            pl.BlockSpec((4 * h, h), lambda i: (0, 0)),
            pl.BlockSpec((1, h), lambda i: (0, 0)),
            pl.BlockSpec((1, h), lambda i: (0, 0)),
            pl.BlockSpec((h, 2 * h), lambda i: (0, 0)),
            pl.BlockSpec((h, 2 * h), lambda i: (0, 0)),
            pl.BlockSpec((1, 2 * h), lambda i: (0, 0)),
            pl.BlockSpec((2 * h, h), lambda i: (0, 0)),
            pl.BlockSpec((1, h), lambda i: (0, 0)),
        ],
        out_specs=pl.BlockSpec((bn, h), lambda i: (i, 0)),
        out_shape=jax.ShapeDtypeStruct((n, h), jnp.float32),
    )(s0p, s1p, c0p, c1p, node_states, W2ab, b2a, b2b,
      U1a, U1b, u1r, U2t, u2r)


def kernel(node_states, relations, W1, b1, W2, b2, U1, u1, U2, u2):
    n, h = node_states.shape
    pairs = relations.reshape(-1, 2)
    ntup = pairs.shape[0]
    nchunks = ntup // CH
    pad = 8
    i0r = jnp.concatenate(
        [pairs[:, 0].reshape(nchunks, CH),
         jnp.zeros((pad, CH), jnp.int32)], axis=0)
    i1r = jnp.concatenate(
        [pairs[:, 1].reshape(nchunks, CH),
         jnp.zeros((pad, CH), jnp.int32)], axis=0)

    A = W1[:, :h].T
    B = W1[:, h:].T
    Wcat = jnp.concatenate([A, B], axis=1)
    bcat = jnp.concatenate([b1, jnp.zeros_like(b1)]).reshape(1, 4 * h)
    W2ab = jnp.concatenate([W2[:h].T, W2[h:].T], axis=0)
    b2a = b2[:h].reshape(1, h)
    b2b = b2[h:].reshape(1, h)
    U1a = U1[:, :h].T
    U1b = U1[:, h:].T
    u1r = u1.reshape(1, 2 * h)
    U2t = U2.T
    u2r = u2.reshape(1, h)

    z64 = jnp.zeros(((n // NS) // 8 * 8, WQ), jnp.float32)
    z16 = jnp.zeros(((n // NS) // 8 * 8, 16), jnp.float32)
    one16 = jnp.ones((CH, 16), jnp.float32)

    tabs = _tc_tables(node_states, Wcat, bcat)
    ptabs, qtabs = tabs[:KQ], tabs[KQ:]
    c0p, c1p = _sc_counts(i0r, i1r, z16, one16, n, nchunks)
    s0p, s1p = _sc_scatter(ptabs, qtabs, i0r, i1r, z64, n, nchunks)
    return _tc_update(s0p, s1p, c0p, c1p, node_states, W2ab, b2a, b2b,
                      U1a, U1b, u1r, U2t, u2r)


# speedup vs baseline: 3.9564x; 1.1504x over previous
import functools
import jax
import jax.numpy as jnp
from jax import lax
from jax.experimental import pallas as pl
from jax.experimental.pallas import tpu as pltpu
from jax.experimental.pallas import tpu_sc as plsc

NC = 2
NS = 16
NW = NC * NS
CH = 64
CI = 2 * CH
KQ = 4
WQ = 64


def _tc_tables(node_states, Wcat, bcat):
    n, h = node_states.shape
    bn = 1000

    def body(ns_ref, w_ref, b_ref, *outs):
        pq = jnp.dot(ns_ref[...], w_ref[...],
                     preferred_element_type=jnp.float32) + b_ref[...]
        for q in range(2 * KQ):
            outs[q][...] = pq[:, WQ * q:WQ * (q + 1)]

    return pl.pallas_call(
        body,
        grid=(n // bn,),
        in_specs=[
            pl.BlockSpec((bn, h), lambda i: (i, 0)),
            pl.BlockSpec((h, 4 * h), lambda i: (0, 0)),
            pl.BlockSpec((1, 4 * h), lambda i: (0, 0)),
        ],
        out_specs=[pl.BlockSpec((bn, WQ), lambda i: (i, 0))
                   for _ in range(2 * KQ)],
        out_shape=[jax.ShapeDtypeStruct((n, WQ), jnp.float32)
                   for _ in range(2 * KQ)],
    )(node_states, Wcat, bcat)


def _sc_scatter(tabs, rel2, z64, z16, one16, n, nchunks):
    n2 = 2 * n
    per_w = nchunks // NW
    rem = nchunks - per_w * NW
    njmax = per_w + (1 if rem else 0)
    rows = (n2 // NS) // 8 * 8
    tail = n2 - rows * NS

    mesh = plsc.VectorSubcoreMesh(core_axis_name="c", subcore_axis_name="s",
                                  num_cores=NC, num_subcores=NS)

    @functools.partial(
        pl.kernel,
        out_type=[
            jax.ShapeDtypeStruct((KQ, NC, n2, WQ), jnp.float32),
            jax.ShapeDtypeStruct((NC, n2, 16), jnp.float32),
        ],
        mesh=mesh,
        scratch_types=[
            pltpu.VMEM_SHARED((n2, WQ), jnp.float32),
            pltpu.VMEM_SHARED((n2, 16), jnp.float32),
            pltpu.VMEM((njmax, CI), jnp.int32),
            pltpu.VMEM((2, CI, WQ), jnp.float32),
            pltpu.VMEM((CI, 16), jnp.float32),
            pltpu.SemaphoreType.DMA,
            pltpu.SemaphoreType.DMA,
        ],
        compiler_params=pltpu.CompilerParams(use_tc_tiling_on_sc=False),
    )
    def k(t0, t1, t2, t3, rel_h, z64_h, z16_h, one16_h,
          s_out, c_out,
          s_sh, c_sh, idx, buf, ones, sem0, sem1):
        tab = (t0, t1, t2, t3)
        sems = (sem0, sem1)
        cid = lax.axis_index("c")
        sid = lax.axis_index("s")
        wid = sid * NC + cid
        nj = per_w + (wid < rem).astype(jnp.int32)
        row0 = wid * per_w + jnp.minimum(wid, rem)
        row0p = jnp.minimum(row0, nchunks - njmax)
        off = row0 - row0p
        pltpu.sync_copy(one16_h, ones)
        pltpu.sync_copy(rel_h.at[pl.ds(row0p, njmax)], idx)
        altn = (lax.iota(jnp.int32, 16) % 2) * n

        def remap(r, carry):
            for q in range(CI // 16):
                sl = pl.ds(q * 16, 16)
                idx[r, sl] = idx[r, sl] + altn
            return carry

        lax.fori_loop(0, njmax, remap, 0)

        for kq in range(KQ):
            sl = pl.ds(sid * rows, rows)
            tl = pl.ds(NS * rows, tail)
            pltpu.sync_copy(z64_h, s_sh.at[sl])
            if kq == 0:
                pltpu.sync_copy(z16_h, c_sh.at[sl])

            @pl.when(sid == NS - 1)
            def _():
                pltpu.sync_copy(z64_h.at[pl.ds(0, tail)], s_sh.at[tl])
                if kq == 0:
                    pltpu.sync_copy(z16_h.at[pl.ds(0, tail)], c_sh.at[tl])

            plsc.subcore_barrier()

            def issue(j, b):
                pltpu.async_copy(tab[kq].at[idx.at[j + off]], buf.at[b],
                                 sems[b])

            def process(j, b):
                pltpu.make_async_copy(tab[kq].at[idx.at[j + off]], buf.at[b],
                                      sems[b]).wait()

                def row(t, carry):
                    r = 2 * t
                    for q in range(WQ // 16):
                        qs = pl.ds(q * 16, 16)
                        m = jnp.maximum(buf[b, r, qs] + buf[b, r + 1, qs],
                                        0.0)
                        buf[b, r, qs] = m
                        buf[b, r + 1, qs] = m
                    return carry

                lax.fori_loop(0, CH, row, 0)
                pltpu.sync_copy(buf.at[b], s_sh.at[idx.at[j + off]],
                                add=True)
                if kq == 0:
                    pltpu.sync_copy(ones, c_sh.at[idx.at[j + off]],
                                    add=True)

            def pair(jj, carry):
                j0 = 2 * jj
                j1 = j0 + 1

                @pl.when(j1 < nj)
                def _():
                    issue(j1, 1)

                process(j0, 0)

                @pl.when(j1 + 1 < nj)
                def _():
                    issue(j1 + 1, 0)

                @pl.when(j1 < nj)
                def _():
                    process(j1, 1)

                return carry

            issue(0, 0)
            lax.fori_loop(0, (nj + 1) // 2, pair, 0)
            plsc.subcore_barrier()

            pltpu.sync_copy(s_sh.at[sl], s_out.at[kq, cid, sl])
            if kq == 0:
                pltpu.sync_copy(c_sh.at[sl], c_out.at[cid, sl])

            @pl.when(sid == NS - 1)
            def _():
                pltpu.sync_copy(s_sh.at[tl], s_out.at[kq, cid, tl])
                if kq == 0:
                    pltpu.sync_copy(c_sh.at[tl], c_out.at[cid, tl])

            plsc.subcore_barrier()

    return k(*tabs, rel2, z64, z16, one16)


def _tc_update(sp, cp, node_states, W2ab, b2a, b2b, U1a, U1b, u1r, U2t, u2r):
    n, h = node_states.shape
    bn = 1000
    nb = n // bn

    def body(s0_ref, s1_ref, c0_ref, c1_ref, ns_ref, w2_ref, b2a_ref, b2b_ref,
             u1a_ref, u1b_ref, u1_ref, u2t_ref, u2_ref, out_ref):
        s0 = jnp.concatenate(
            [s0_ref[kq, 0] + s0_ref[kq, 1] for kq in range(KQ)], axis=1)
        s1 = jnp.concatenate(
            [s1_ref[kq, 0] + s1_ref[kq, 1] for kq in range(KQ)], axis=1)
        c0 = c0_ref[0, :, 0] + c0_ref[1, :, 0]
        c1 = c1_ref[0, :, 0] + c1_ref[1, :, 0]
        s01 = jnp.concatenate([s0, s1], axis=1)
        summ = jnp.dot(s01, w2_ref[...], preferred_element_type=jnp.float32)
        summ = summ + c0[:, None] * b2a_ref[...] + c1[:, None] * b2b_ref[...]
        z = jnp.maximum(
            jnp.dot(summ, u1a_ref[...], preferred_element_type=jnp.float32)
            + jnp.dot(ns_ref[...], u1b_ref[...],
                      preferred_element_type=jnp.float32)
            + u1_ref[...], 0.0)
        out_ref[...] = (jnp.dot(z, u2t_ref[...],
                                preferred_element_type=jnp.float32)
                        + u2_ref[...])

    return pl.pallas_call(
        body,
        grid=(nb,),
        in_specs=[
            pl.BlockSpec((KQ, NC, bn, WQ), lambda i: (0, 0, i, 0)),
            pl.BlockSpec((KQ, NC, bn, WQ), lambda i: (0, 0, i + nb, 0)),
            pl.BlockSpec((NC, bn, 16), lambda i: (0, i, 0)),
            pl.BlockSpec((NC, bn, 16), lambda i: (0, i + nb, 0)),
            pl.BlockSpec((bn, h), lambda i: (i, 0)),
            pl.BlockSpec((4 * h, h), lambda i: (0, 0)),
            pl.BlockSpec((1, h), lambda i: (0, 0)),
            pl.BlockSpec((1, h), lambda i: (0, 0)),
            pl.BlockSpec((h, 2 * h), lambda i: (0, 0)),
            pl.BlockSpec((h, 2 * h), lambda i: (0, 0)),
            pl.BlockSpec((1, 2 * h), lambda i: (0, 0)),
            pl.BlockSpec((2 * h, h), lambda i: (0, 0)),
            pl.BlockSpec((1, h), lambda i: (0, 0)),
        ],
        out_specs=pl.BlockSpec((bn, h), lambda i: (i, 0)),
        out_shape=jax.ShapeDtypeStruct((n, h), jnp.float32),
    )(sp, sp, cp, cp, node_states, W2ab, b2a, b2b, U1a, U1b, u1r, U2t, u2r)


def kernel(node_states, relations, W1, b1, W2, b2, U1, u1, U2, u2):
    n, h = node_states.shape
    nchunks = relations.shape[0] // CI
    rel2 = relations.reshape(nchunks, CI)

    A = W1[:, :h].T
    B = W1[:, h:].T
    Wcat = jnp.concatenate([A, B], axis=1)
    bcat = jnp.concatenate([b1, jnp.zeros_like(b1)]).reshape(1, 4 * h)
    W2ab = jnp.concatenate([W2[:h].T, W2[h:].T], axis=0)
    b2a = b2[:h].reshape(1, h)
    b2b = b2[h:].reshape(1, h)
    U1a = U1[:, :h].T
    U1b = U1[:, h:].T
    u1r = u1.reshape(1, 2 * h)
    U2t = U2.T
    u2r = u2.reshape(1, h)

    zrows = ((2 * n) // NS) // 8 * 8
    z64 = jnp.zeros((zrows, WQ), jnp.float32)
    z16 = jnp.zeros((zrows, 16), jnp.float32)
    one16 = jnp.ones((CI, 16), jnp.float32)

    pq = _tc_tables(node_states, Wcat, bcat)
    tabs = [jnp.concatenate([pq[k], pq[KQ + k]], axis=0) for k in range(KQ)]
    sp, cp = _sc_scatter(tabs, rel2, z64, z16, one16, n, nchunks)
    return _tc_update(sp, cp, node_states, W2ab, b2a, b2b,
                      U1a, U1b, u1r, U2t, u2r)


# speedup vs baseline: 5.7662x; 1.4574x over previous
import functools
import jax
import jax.numpy as jnp
from jax import lax
from jax.experimental import pallas as pl
from jax.experimental.pallas import tpu as pltpu
from jax.experimental.pallas import tpu_sc as plsc

NC = 2
NS = 16
NW = NC * NS
CH = 64
CI = 2 * CH
KQ = 2
WQ = 128


def _tc_tables(node_states, Wcat, bcat):
    n, h = node_states.shape
    bn = 1000
    nb = n // bn

    def body(ns_ref, w_ref, b_ref, outa, outb):
        half = jnp.dot(ns_ref[...], w_ref[...],
                       preferred_element_type=jnp.float32) + b_ref[...]
        hb = half.astype(jnp.bfloat16)
        outa[...] = hb[:, :WQ]
        outb[...] = hb[:, WQ:]

    return pl.pallas_call(
        body,
        grid=(2, nb),
        in_specs=[
            pl.BlockSpec((bn, h), lambda g, i: (i, 0)),
            pl.BlockSpec((h, 2 * h), lambda g, i: (0, g)),
            pl.BlockSpec((1, 2 * h), lambda g, i: (0, g)),
        ],
        out_specs=[pl.BlockSpec((bn, WQ), lambda g, i: (g * nb + i, 0)),
                   pl.BlockSpec((bn, WQ), lambda g, i: (g * nb + i, 0))],
        out_shape=[jax.ShapeDtypeStruct((2 * n, WQ), jnp.bfloat16),
                   jax.ShapeDtypeStruct((2 * n, WQ), jnp.bfloat16)],
    )(node_states, Wcat, bcat)


def _sc_scatter(tabs, rel2, zb, z16, one16, n, nchunks):
    n2 = 2 * n
    per_w = nchunks // NW
    rem = nchunks - per_w * NW
    njmax = per_w + (1 if rem else 0)
    rows = (n2 // NS) // 8 * 8
    tail = n2 - rows * NS

    mesh = plsc.VectorSubcoreMesh(core_axis_name="c", subcore_axis_name="s",
                                  num_cores=NC, num_subcores=NS)

    @functools.partial(
        pl.kernel,
        out_type=[
            jax.ShapeDtypeStruct((KQ, NC, n2, WQ), jnp.bfloat16),
            jax.ShapeDtypeStruct((NC, n2, 16), jnp.float32),
        ],
        mesh=mesh,
        scratch_types=[
            pltpu.VMEM_SHARED((n2, WQ), jnp.bfloat16),
            pltpu.VMEM_SHARED((n2, 16), jnp.float32),
            pltpu.VMEM((njmax, CI), jnp.int32),
            pltpu.VMEM((2, CI, WQ), jnp.bfloat16),
            pltpu.VMEM((CI, 16), jnp.float32),
            pltpu.SemaphoreType.DMA,
            pltpu.SemaphoreType.DMA,
        ],
        compiler_params=pltpu.CompilerParams(use_tc_tiling_on_sc=False),
    )
    def k(t0, t1, rel_h, zb_h, z16_h, one16_h,
          s_out, c_out,
          s_sh, c_sh, idx, buf, ones, sem0, sem1):
        tab = (t0, t1)
        sems = (sem0, sem1)
        cid = lax.axis_index("c")
        sid = lax.axis_index("s")
        wid = sid * NC + cid
        nj = per_w + (wid < rem).astype(jnp.int32)
        row0 = wid * per_w + jnp.minimum(wid, rem)
        row0p = jnp.minimum(row0, nchunks - njmax)
        off = row0 - row0p
        pltpu.sync_copy(one16_h, ones)
        pltpu.sync_copy(rel_h.at[pl.ds(row0p, njmax)], idx)
        altn = (lax.iota(jnp.int32, 16) % 2) * n

        def remap(r, carry):
            for q in range(CI // 16):
                sl = pl.ds(q * 16, 16)
                idx[r, sl] = idx[r, sl] + altn
            return carry

        lax.fori_loop(0, njmax, remap, 0)

        for kq in range(KQ):
            sl = pl.ds(sid * rows, rows)
            tl = pl.ds(NS * rows, tail)
            pltpu.sync_copy(zb_h, s_sh.at[sl])
            if kq == 0:
                pltpu.sync_copy(z16_h, c_sh.at[sl])

            @pl.when(sid == NS - 1)
            def _():
                pltpu.sync_copy(zb_h.at[pl.ds(0, tail)], s_sh.at[tl])
                if kq == 0:
                    pltpu.sync_copy(z16_h.at[pl.ds(0, tail)], c_sh.at[tl])

            plsc.subcore_barrier()

            def issue(j, b):
                pltpu.async_copy(tab[kq].at[idx.at[j + off]], buf.at[b],
                                 sems[b])

            def process(j, b):
                pltpu.make_async_copy(tab[kq].at[idx.at[j + off]], buf.at[b],
                                      sems[b]).wait()

                def row(t, carry):
                    r = 2 * t
                    for q in range(WQ // 32):
                        qs = pl.ds(q * 32, 32)
                        m = jnp.maximum(buf[b, r, qs] + buf[b, r + 1, qs],
                                        jnp.bfloat16(0.0))
                        buf[b, r, qs] = m
                        buf[b, r + 1, qs] = m
                    return carry

                lax.fori_loop(0, CH, row, 0)
                pltpu.sync_copy(buf.at[b], s_sh.at[idx.at[j + off]],
                                add=True)
                if kq == 0:
                    pltpu.sync_copy(ones, c_sh.at[idx.at[j + off]],
                                    add=True)

            def pair(jj, carry):
                j0 = 2 * jj
                j1 = j0 + 1

                @pl.when(j1 < nj)
                def _():
                    issue(j1, 1)

                process(j0, 0)

                @pl.when(j1 + 1 < nj)
                def _():
                    issue(j1 + 1, 0)

                @pl.when(j1 < nj)
                def _():
                    process(j1, 1)

                return carry

            issue(0, 0)
            lax.fori_loop(0, (nj + 1) // 2, pair, 0)
            plsc.subcore_barrier()

            pltpu.sync_copy(s_sh.at[sl], s_out.at[kq, cid, sl])
            if kq == 0:
                pltpu.sync_copy(c_sh.at[sl], c_out.at[cid, sl])

            @pl.when(sid == NS - 1)
            def _():
                pltpu.sync_copy(s_sh.at[tl], s_out.at[kq, cid, tl])
                if kq == 0:
                    pltpu.sync_copy(c_sh.at[tl], c_out.at[cid, tl])

            plsc.subcore_barrier()

    return k(*tabs, rel2, zb, z16, one16)


def _tc_update(sp, cp, node_states, W2ab, b2a, b2b, U1a, U1b, u1r, U2t, u2r):
    n, h = node_states.shape
    bn = 1000
    nb = n // bn

    def body(s0_ref, s1_ref, c0_ref, c1_ref, ns_ref, w2_ref, b2a_ref, b2b_ref,
             u1a_ref, u1b_ref, u1_ref, u2t_ref, u2_ref, out_ref):
        s0 = jnp.concatenate(
            [(s0_ref[kq, 0] + s0_ref[kq, 1]).astype(jnp.float32)
             for kq in range(KQ)], axis=1)
        s1 = jnp.concatenate(
            [(s1_ref[kq, 0] + s1_ref[kq, 1]).astype(jnp.float32)
             for kq in range(KQ)], axis=1)
        c0 = c0_ref[0, :, 0] + c0_ref[1, :, 0]
        c1 = c1_ref[0, :, 0] + c1_ref[1, :, 0]
        s01 = jnp.concatenate([s0, s1], axis=1)
        summ = jnp.dot(s01, w2_ref[...], preferred_element_type=jnp.float32)
        summ = summ + c0[:, None] * b2a_ref[...] + c1[:, None] * b2b_ref[...]
        z = jnp.maximum(
            jnp.dot(summ, u1a_ref[...], preferred_element_type=jnp.float32)
            + jnp.dot(ns_ref[...], u1b_ref[...],
                      preferred_element_type=jnp.float32)
            + u1_ref[...], 0.0)
        out_ref[...] = (jnp.dot(z, u2t_ref[...],
                                preferred_element_type=jnp.float32)
                        + u2_ref[...])

    return pl.pallas_call(
        body,
        grid=(nb,),
        in_specs=[
            pl.BlockSpec((KQ, NC, bn, WQ), lambda i: (0, 0, i, 0)),
            pl.BlockSpec((KQ, NC, bn, WQ), lambda i: (0, 0, i + nb, 0)),
            pl.BlockSpec((NC, bn, 16), lambda i: (0, i, 0)),
            pl.BlockSpec((NC, bn, 16), lambda i: (0, i + nb, 0)),
            pl.BlockSpec((bn, h), lambda i: (i, 0)),
            pl.BlockSpec((4 * h, h), lambda i: (0, 0)),
            pl.BlockSpec((1, h), lambda i: (0, 0)),
            pl.BlockSpec((1, h), lambda i: (0, 0)),
            pl.BlockSpec((h, 2 * h), lambda i: (0, 0)),
            pl.BlockSpec((h, 2 * h), lambda i: (0, 0)),
            pl.BlockSpec((1, 2 * h), lambda i: (0, 0)),
            pl.BlockSpec((2 * h, h), lambda i: (0, 0)),
            pl.BlockSpec((1, h), lambda i: (0, 0)),
        ],
        out_specs=pl.BlockSpec((bn, h), lambda i: (i, 0)),
        out_shape=jax.ShapeDtypeStruct((n, h), jnp.float32),
    )(sp, sp, cp, cp, node_states, W2ab, b2a, b2b, U1a, U1b, u1r, U2t, u2r)


def kernel(node_states, relations, W1, b1, W2, b2, U1, u1, U2, u2):
    n, h = node_states.shape
    nchunks = relations.shape[0] // CI
    rel2 = relations.reshape(nchunks, CI)

    A = W1[:, :h].T
    B = W1[:, h:].T
    Wcat = jnp.concatenate([A, B], axis=1)
    bcat = jnp.concatenate([b1, jnp.zeros_like(b1)]).reshape(1, 4 * h)
    W2ab = jnp.concatenate([W2[:h].T, W2[h:].T], axis=0)
    b2a = b2[:h].reshape(1, h)
    b2b = b2[h:].reshape(1, h)
    U1a = U1[:, :h].T
    U1b = U1[:, h:].T
    u1r = u1.reshape(1, 2 * h)
    U2t = U2.T
    u2r = u2.reshape(1, h)

    zrows = ((2 * n) // NS) // 8 * 8
    zb = jnp.zeros((zrows, WQ), jnp.bfloat16)
    z16 = jnp.zeros((zrows, 16), jnp.float32)
    one16 = jnp.ones((CI, 16), jnp.float32)

    tabs = _tc_tables(node_states, Wcat, bcat)
    sp, cp = _sc_scatter(tabs, rel2, zb, z16, one16, n, nchunks)
    return _tc_update(sp, cp, node_states, W2ab, b2a, b2b,
                      U1a, U1b, u1r, U2t, u2r)


# speedup vs baseline: 6.1800x; 1.0718x over previous
import functools
import jax
import jax.numpy as jnp
from jax import lax
from jax.experimental import pallas as pl
from jax.experimental.pallas import tpu as pltpu
from jax.experimental.pallas import tpu_sc as plsc

NC = 2
NS = 16
NW = NC * NS
CH = 64
CI = 2 * CH
KQ = 2
WQ = 128


def _tc_tables(node_states, Wcat, bcat):
    n, h = node_states.shape
    bn = 1000
    nb = n // bn

    def body(ns_ref, w_ref, b_ref, outa, outb):
        half = jnp.dot(ns_ref[...], w_ref[...],
                       preferred_element_type=jnp.float32) + b_ref[...]
        hb = half.astype(jnp.bfloat16)
        outa[...] = hb[:, :WQ]
        outb[...] = hb[:, WQ:]

    return pl.pallas_call(
        body,
        grid=(2, nb),
        in_specs=[
            pl.BlockSpec((bn, h), lambda g, i: (i, 0)),
            pl.BlockSpec((h, 2 * h), lambda g, i: (0, g)),
            pl.BlockSpec((1, 2 * h), lambda g, i: (0, g)),
        ],
        out_specs=[pl.BlockSpec((bn, WQ), lambda g, i: (g * nb + i, 0)),
                   pl.BlockSpec((bn, WQ), lambda g, i: (g * nb + i, 0))],
        out_shape=[jax.ShapeDtypeStruct((2 * n, WQ), jnp.bfloat16),
                   jax.ShapeDtypeStruct((2 * n, WQ), jnp.bfloat16)],
    )(node_states, Wcat, bcat)


def _sc_scatter(tabs, rel2, zb, z16, one16, n, nchunks):
    n2 = 2 * n
    per_w = nchunks // NW
    rem = nchunks - per_w * NW
    njmax = per_w + (1 if rem else 0)
    rows = (n2 // NS) // 8 * 8
    tail = n2 - rows * NS

    mesh = plsc.VectorSubcoreMesh(core_axis_name="c", subcore_axis_name="s",
                                  num_cores=NC, num_subcores=NS)

    hj = (njmax + 1) // 2

    @functools.partial(
        pl.kernel,
        out_type=[
            jax.ShapeDtypeStruct((KQ, NC, n2, WQ), jnp.bfloat16),
            jax.ShapeDtypeStruct((NC, n2, 16), jnp.bfloat16),
        ],
        mesh=mesh,
        scratch_types=[
            pltpu.VMEM_SHARED((n2, WQ), jnp.bfloat16),
            pltpu.VMEM_SHARED((n2, 16), jnp.bfloat16),
            pltpu.VMEM((hj, CI), jnp.int32),
            pltpu.VMEM((2, CI, WQ), jnp.bfloat16),
            pltpu.VMEM((2, CI, WQ), jnp.bfloat16),
            pltpu.VMEM((CI, 16), jnp.bfloat16),
            pltpu.SemaphoreType.DMA,
            pltpu.SemaphoreType.DMA,
            pltpu.SemaphoreType.DMA,
            pltpu.SemaphoreType.DMA,
            pltpu.SemaphoreType.DMA,
        ],
        compiler_params=pltpu.CompilerParams(use_tc_tiling_on_sc=False),
    )
    def k(t0, t1, rel_h, zb_h, z16_h, one16_h,
          s_out, c_out,
          s_sh, c_sh, idx, gbuf, sbuf, ones,
          gsem0, gsem1, ssem0, ssem1, csem):
        tab = (t0, t1)
        gsems = (gsem0, gsem1)
        ssems = (ssem0, ssem1)
        cid = lax.axis_index("c")
        sid = lax.axis_index("s")
        wid = sid * NC + cid
        nj = per_w + (wid < rem).astype(jnp.int32)
        row0 = wid * per_w + jnp.minimum(wid, rem)
        pltpu.sync_copy(one16_h, ones)
        altn = (lax.iota(jnp.int32, 16) % 2) * n

        for kq in range(KQ):
            sl = pl.ds(sid * rows, rows)
            tl = pl.ds(NS * rows, tail)
            pltpu.sync_copy(zb_h, s_sh.at[sl])
            if kq == 0:
                pltpu.sync_copy(z16_h, c_sh.at[sl])

            @pl.when(sid == NS - 1)
            def _():
                pltpu.sync_copy(zb_h.at[pl.ds(0, tail)], s_sh.at[tl])
                if kq == 0:
                    pltpu.sync_copy(z16_h.at[pl.ds(0, tail)], c_sh.at[tl])

            plsc.subcore_barrier()

            for hh in range(2):
                gstart = row0 + hh * hj
                hn = jnp.clip(nj - hh * hj, 0, hj)
                lbase = jnp.minimum(gstart, nchunks - hj)
                off = gstart - lbase
                pltpu.sync_copy(rel_h.at[pl.ds(lbase, hj)], idx)

                def remap(r, carry):
                    for q in range(CI // 16):
                        qs = pl.ds(q * 16, 16)
                        idx[r, qs] = idx[r, qs] + altn
                    return carry

                lax.fori_loop(0, hj, remap, 0)

                def issue(j, b):
                    pltpu.async_copy(tab[kq].at[idx.at[j + off]],
                                     gbuf.at[b], gsems[b])

                def wait_scatter(b, j):
                    pltpu.make_async_copy(sbuf.at[b],
                                          s_sh.at[idx.at[j + off]],
                                          ssems[b]).wait()

                def process(j, b):
                    pltpu.make_async_copy(tab[kq].at[idx.at[j + off]],
                                          gbuf.at[b], gsems[b]).wait()

                    @pl.when(j >= 2)
                    def _():
                        wait_scatter(b, j)

                    def row(t, carry):
                        r = 2 * t
                        for q in range(WQ // 32):
                            qs = pl.ds(q * 32, 32)
                            m = jnp.maximum(
                                gbuf[b, r, qs] + gbuf[b, r + 1, qs],
                                jnp.bfloat16(0.0))
                            sbuf[b, r, qs] = m
                            sbuf[b, r + 1, qs] = m
                        return carry

                    lax.fori_loop(0, CH, row, 0)
                    pltpu.async_copy(sbuf.at[b], s_sh.at[idx.at[j + off]],
                                     ssems[b], add=True)
                    if kq == 0:
                        pltpu.async_copy(ones, c_sh.at[idx.at[j + off]],
                                         csem, add=True)

                    @pl.when(j + 2 < hn)
                    def _():
                        issue(j + 2, b)

                def pair(jj, carry):
                    j0 = 2 * jj
                    process(j0, 0)

                    @pl.when(j0 + 1 < hn)
                    def _():
                        process(j0 + 1, 1)

                    return carry

                issue(0, 0)

                @pl.when(hn > 1)
                def _():
                    issue(1, 1)

                lax.fori_loop(0, (hn + 1) // 2, pair, 0)

                @pl.when(hn >= 1)
                def _():
                    wait_scatter(0, 0)

                @pl.when(hn >= 2)
                def _():
                    wait_scatter(1, 0)

                if kq == 0:
                    def cdrain(j, carry):
                        pltpu.make_async_copy(ones,
                                              c_sh.at[idx.at[0]],
                                              csem).wait()
                        return carry

                    lax.fori_loop(0, hn, cdrain, 0)

            plsc.subcore_barrier()

            pltpu.sync_copy(s_sh.at[sl], s_out.at[kq, cid, sl])
            if kq == 0:
                pltpu.sync_copy(c_sh.at[sl], c_out.at[cid, sl])

            @pl.when(sid == NS - 1)
            def _():
                pltpu.sync_copy(s_sh.at[tl], s_out.at[kq, cid, tl])
                if kq == 0:
                    pltpu.sync_copy(c_sh.at[tl], c_out.at[cid, tl])

            plsc.subcore_barrier()

    return k(*tabs, rel2, zb, z16, one16)


def _tc_update(sp, cp, node_states, W2ab, b2a, b2b, U1a, U1b, u1r, U2t, u2r):
    n, h = node_states.shape
    bn = 1000
    nb = n // bn

    def body(s0_ref, s1_ref, c0_ref, c1_ref, ns_ref, w2_ref, b2a_ref, b2b_ref,
             u1a_ref, u1b_ref, u1_ref, u2t_ref, u2_ref, out_ref):
        s0 = jnp.concatenate(
            [(s0_ref[kq, 0] + s0_ref[kq, 1]).astype(jnp.float32)
             for kq in range(KQ)], axis=1)
        s1 = jnp.concatenate(
            [(s1_ref[kq, 0] + s1_ref[kq, 1]).astype(jnp.float32)
             for kq in range(KQ)], axis=1)
        c0 = (c0_ref[0, :, 0] + c0_ref[1, :, 0]).astype(jnp.float32)
        c1 = (c1_ref[0, :, 0] + c1_ref[1, :, 0]).astype(jnp.float32)
        s01 = jnp.concatenate([s0, s1], axis=1)
        summ = jnp.dot(s01, w2_ref[...], preferred_element_type=jnp.float32)
        summ = summ + c0[:, None] * b2a_ref[...] + c1[:, None] * b2b_ref[...]
        z = jnp.maximum(
            jnp.dot(summ, u1a_ref[...], preferred_element_type=jnp.float32)
            + jnp.dot(ns_ref[...], u1b_ref[...],
                      preferred_element_type=jnp.float32)
            + u1_ref[...], 0.0)
        out_ref[...] = (jnp.dot(z, u2t_ref[...],
                                preferred_element_type=jnp.float32)
                        + u2_ref[...])

    return pl.pallas_call(
        body,
        grid=(nb,),
        in_specs=[
            pl.BlockSpec((KQ, NC, bn, WQ), lambda i: (0, 0, i, 0)),
            pl.BlockSpec((KQ, NC, bn, WQ), lambda i: (0, 0, i + nb, 0)),
            pl.BlockSpec((NC, bn, 16), lambda i: (0, i, 0)),
            pl.BlockSpec((NC, bn, 16), lambda i: (0, i + nb, 0)),
            pl.BlockSpec((bn, h), lambda i: (i, 0)),
            pl.BlockSpec((4 * h, h), lambda i: (0, 0)),
            pl.BlockSpec((1, h), lambda i: (0, 0)),
            pl.BlockSpec((1, h), lambda i: (0, 0)),
            pl.BlockSpec((h, 2 * h), lambda i: (0, 0)),
            pl.BlockSpec((h, 2 * h), lambda i: (0, 0)),
            pl.BlockSpec((1, 2 * h), lambda i: (0, 0)),
            pl.BlockSpec((2 * h, h), lambda i: (0, 0)),
            pl.BlockSpec((1, h), lambda i: (0, 0)),
        ],
        out_specs=pl.BlockSpec((bn, h), lambda i: (i, 0)),
        out_shape=jax.ShapeDtypeStruct((n, h), jnp.float32),
    )(sp, sp, cp, cp, node_states, W2ab, b2a, b2b, U1a, U1b, u1r, U2t, u2r)


def kernel(node_states, relations, W1, b1, W2, b2, U1, u1, U2, u2):
    n, h = node_states.shape
    nchunks = relations.shape[0] // CI
    rel2 = relations.reshape(nchunks, CI)

    A = W1[:, :h].T
    B = W1[:, h:].T
    Wcat = jnp.concatenate([A, B], axis=1)
    bcat = jnp.concatenate([b1, jnp.zeros_like(b1)]).reshape(1, 4 * h)
    W2ab = jnp.concatenate([W2[:h].T, W2[h:].T], axis=0)
    b2a = b2[:h].reshape(1, h)
    b2b = b2[h:].reshape(1, h)
    U1a = U1[:, :h].T
    U1b = U1[:, h:].T
    u1r = u1.reshape(1, 2 * h)
    U2t = U2.T
    u2r = u2.reshape(1, h)

    zrows = ((2 * n) // NS) // 8 * 8
    zb = jnp.zeros((zrows, WQ), jnp.bfloat16)
    z16 = jnp.zeros((zrows, 16), jnp.bfloat16)
    one16 = jnp.ones((CI, 16), jnp.bfloat16)

    tabs = _tc_tables(node_states, Wcat, bcat)
    sp, cp = _sc_scatter(tabs, rel2, zb, z16, one16, n, nchunks)
    return _tc_update(sp, cp, node_states, W2ab, b2a, b2b,
                      U1a, U1b, u1r, U2t, u2r)


# speedup vs baseline: 7.0412x; 1.1393x over previous
import functools
import jax
import jax.numpy as jnp
from jax import lax
from jax.experimental import pallas as pl
from jax.experimental.pallas import tpu as pltpu
from jax.experimental.pallas import tpu_sc as plsc

NC = 2
NS = 16
NW = NC * NS
CH = 64
CI = 2 * CH
KQ = 2
WQ = 128


def _tc_tables(node_states, Wcat, bcat):
    n, h = node_states.shape
    bn = 1000
    nb = n // bn

    def body(ns_ref, w_ref, b_ref, outa, outb):
        half = jnp.dot(ns_ref[...], w_ref[...],
                       preferred_element_type=jnp.float32) + b_ref[...]
        hb = half.astype(jnp.bfloat16)
        outa[...] = hb[:, :WQ]
        outb[...] = hb[:, WQ:]

    return pl.pallas_call(
        body,
        grid=(2, nb),
        in_specs=[
            pl.BlockSpec((bn, h), lambda g, i: (i, 0)),
            pl.BlockSpec((h, 2 * h), lambda g, i: (0, g)),
            pl.BlockSpec((1, 2 * h), lambda g, i: (0, g)),
        ],
        out_specs=[pl.BlockSpec((bn, WQ), lambda g, i: (g * nb + i, 0)),
                   pl.BlockSpec((bn, WQ), lambda g, i: (g * nb + i, 0))],
        out_shape=[jax.ShapeDtypeStruct((2 * n, WQ), jnp.bfloat16),
                   jax.ShapeDtypeStruct((2 * n, WQ), jnp.bfloat16)],
    )(node_states, Wcat, bcat)


def _sc_scatter(tabs, rel2, zb, z16, one16, n, nchunks):
    n2 = 2 * n
    per_w = nchunks // NW
    rem = nchunks - per_w * NW
    njmax = per_w + (1 if rem else 0)
    rows = (n2 // NS) // 8 * 8
    tail = n2 - rows * NS

    mesh = plsc.VectorSubcoreMesh(core_axis_name="c", subcore_axis_name="s",
                                  num_cores=NC, num_subcores=NS)

    hj = (njmax + 1) // 2

    @functools.partial(
        pl.kernel,
        out_type=[
            jax.ShapeDtypeStruct((KQ, NC, n2, WQ), jnp.bfloat16),
            jax.ShapeDtypeStruct((NC, n2, 16), jnp.bfloat16),
        ],
        mesh=mesh,
        scratch_types=[
            pltpu.VMEM_SHARED((n2, WQ), jnp.bfloat16),
            pltpu.VMEM_SHARED((n2, 16), jnp.bfloat16),
            pltpu.VMEM((hj, CI), jnp.int32),
            pltpu.VMEM((2, CI, WQ), jnp.bfloat16),
            pltpu.VMEM((2, CI, WQ), jnp.bfloat16),
            pltpu.VMEM((CI, 16), jnp.bfloat16),
            pltpu.SemaphoreType.DMA,
            pltpu.SemaphoreType.DMA,
            pltpu.SemaphoreType.DMA,
            pltpu.SemaphoreType.DMA,
            pltpu.SemaphoreType.DMA,
        ],
        compiler_params=pltpu.CompilerParams(use_tc_tiling_on_sc=False),
    )
    def k(t0, t1, rel_h, zb_h, z16_h, one16_h,
          s_out, c_out,
          s_sh, c_sh, idx, gbuf, sbuf, ones,
          gsem0, gsem1, ssem0, ssem1, csem):
        tab = (t0, t1)
        gsems = (gsem0, gsem1)
        ssems = (ssem0, ssem1)
        cid = lax.axis_index("c")
        sid = lax.axis_index("s")
        wid = sid * NC + cid
        nj = per_w + (wid < rem).astype(jnp.int32)
        row0 = wid * per_w + jnp.minimum(wid, rem)
        pltpu.sync_copy(one16_h, ones)
        altn = (lax.iota(jnp.int32, 16) % 2) * n

        for kq in range(KQ):
            sl = pl.ds(sid * rows, rows)
            tl = pl.ds(NS * rows, tail)
            pltpu.sync_copy(zb_h, s_sh.at[sl])
            if kq == 0:
                pltpu.sync_copy(z16_h, c_sh.at[sl])

            @pl.when(sid == NS - 1)
            def _():
                pltpu.sync_copy(zb_h.at[pl.ds(0, tail)], s_sh.at[tl])
                if kq == 0:
                    pltpu.sync_copy(z16_h.at[pl.ds(0, tail)], c_sh.at[tl])

            plsc.subcore_barrier()

            for hh in range(2):
                gstart = row0 + hh * hj
                hn = jnp.clip(nj - hh * hj, 0, hj)
                lbase = jnp.minimum(gstart, nchunks - hj)
                off = gstart - lbase
                pltpu.sync_copy(rel_h.at[pl.ds(lbase, hj)], idx)

                def remap(r, carry):
                    for q in range(CI // 16):
                        qs = pl.ds(q * 16, 16)
                        idx[r, qs] = idx[r, qs] + altn
                    return carry

                lax.fori_loop(0, hj, remap, 0)

                def issue(j, b):
                    pltpu.async_copy(tab[kq].at[idx.at[j + off]],
                                     gbuf.at[b], gsems[b])

                def wait_scatter(b, j):
                    pltpu.make_async_copy(sbuf.at[b],
                                          s_sh.at[idx.at[j + off]],
                                          ssems[b]).wait()

                def process(j, b):
                    pltpu.make_async_copy(tab[kq].at[idx.at[j + off]],
                                          gbuf.at[b], gsems[b]).wait()

                    @pl.when(j >= 2)
                    def _():
                        wait_scatter(b, j)

                    def row(t, carry):
                        r = 2 * t
                        for q in range(WQ // 32):
                            qs = pl.ds(q * 32, 32)
                            m = jnp.maximum(
                                gbuf[b, r, qs] + gbuf[b, r + 1, qs],
                                jnp.bfloat16(0.0))
                            sbuf[b, r, qs] = m
                            sbuf[b, r + 1, qs] = m
                        return carry

                    lax.fori_loop(0, CH, row, 0)
                    pltpu.async_copy(sbuf.at[b], s_sh.at[idx.at[j + off]],
                                     ssems[b], add=True)
                    if kq == 0:
                        pltpu.async_copy(ones, c_sh.at[idx.at[j + off]],
                                         csem, add=True)

                    @pl.when(j + 2 < hn)
                    def _():
                        issue(j + 2, b)

                def pair(jj, carry):
                    j0 = 2 * jj
                    process(j0, 0)

                    @pl.when(j0 + 1 < hn)
                    def _():
                        process(j0 + 1, 1)

                    return carry

                issue(0, 0)

                @pl.when(hn > 1)
                def _():
                    issue(1, 1)

                lax.fori_loop(0, (hn + 1) // 2, pair, 0)

                @pl.when(hn >= 1)
                def _():
                    wait_scatter(0, 0)

                @pl.when(hn >= 2)
                def _():
                    wait_scatter(1, 0)

                if kq == 0:
                    def cdrain(j, carry):
                        pltpu.make_async_copy(ones,
                                              c_sh.at[idx.at[0]],
                                              csem).wait()
                        return carry

                    lax.fori_loop(0, hn, cdrain, 0)

            plsc.subcore_barrier()

            pltpu.sync_copy(s_sh.at[sl], s_out.at[kq, cid, sl])
            if kq == 0:
                pltpu.sync_copy(c_sh.at[sl], c_out.at[cid, sl])

            @pl.when(sid == NS - 1)
            def _():
                pltpu.sync_copy(s_sh.at[tl], s_out.at[kq, cid, tl])
                if kq == 0:
                    pltpu.sync_copy(c_sh.at[tl], c_out.at[cid, tl])

            plsc.subcore_barrier()

    return k(*tabs, rel2, zb, z16, one16)


def _sc_combine(sp, n):
    n2 = 2 * n
    rows = (n2 // NW) // 8 * 8
    tail = n2 - rows * NW
    CR = 208

    mesh = plsc.VectorSubcoreMesh(core_axis_name="c", subcore_axis_name="s",
                                  num_cores=NC, num_subcores=NS)

    @functools.partial(
        pl.kernel,
        out_type=jax.ShapeDtypeStruct((KQ, n2, WQ), jnp.float32),
        mesh=mesh,
        scratch_types=[
            pltpu.VMEM((CR, WQ), jnp.bfloat16),
            pltpu.VMEM((CR, WQ), jnp.bfloat16),
            pltpu.VMEM((CR, WQ), jnp.float32),
        ],
        compiler_params=pltpu.CompilerParams(use_tc_tiling_on_sc=False,
                                             needs_layout_passes=False),
    )
    def k(sp_h, sf_out, b0, b1, ob):
        cid = lax.axis_index("c")
        sid = lax.axis_index("s")
        wid = sid * NC + cid
        iot = lax.iota(jnp.int32, 16)

        def convert(nrows):
            def rowfn(r, carry):
                rv = jnp.full((16,), r, jnp.int32)
                for q in range(WQ // 32):
                    qs = pl.ds(q * 32, 32)
                    w = plsc.bitcast(b0[r, qs] + b1[r, qs], jnp.int32)
                    fe = plsc.bitcast(w << 16, jnp.float32)
                    fo = plsc.bitcast(w & jnp.int32(-65536), jnp.float32)
                    ce = q * 32 + 2 * iot
                    plsc.store_scatter(ob, [rv, ce], fe)
                    plsc.store_scatter(ob, [rv, ce + 1], fo)
                return carry

            lax.fori_loop(0, nrows, rowfn, 0)

        for kq in range(KQ):
            for ch in range(rows // CR):
                r0 = wid * rows + ch * CR
                pltpu.sync_copy(sp_h.at[kq, 0, pl.ds(r0, CR)], b0)
                pltpu.sync_copy(sp_h.at[kq, 1, pl.ds(r0, CR)], b1)
                convert(CR)
                pltpu.sync_copy(ob, sf_out.at[kq, pl.ds(r0, CR)])

            @pl.when(wid == NW - 1)
            def _():
                r0 = NW * rows
                tb0 = b0.at[pl.ds(0, tail)]
                tb1 = b1.at[pl.ds(0, tail)]
                pltpu.sync_copy(sp_h.at[kq, 0, pl.ds(r0, tail)], tb0)
                pltpu.sync_copy(sp_h.at[kq, 1, pl.ds(r0, tail)], tb1)
                convert(tail)
                pltpu.sync_copy(ob.at[pl.ds(0, tail)],
                                sf_out.at[kq, pl.ds(r0, tail)])

    return k(sp)


def _tc_update(sf, cp, node_states, W2ab, b2a, b2b, U1a, U1b, u1r, U2t, u2r):
    n, h = node_states.shape
    bn = 1000
    nb = n // bn

    def body(s0_ref, s1_ref, c0_ref, c1_ref, ns_ref, w2_ref, b2a_ref, b2b_ref,
             u1a_ref, u1b_ref, u1_ref, u2t_ref, u2_ref, out_ref):
        s0 = jnp.concatenate([s0_ref[kq] for kq in range(KQ)], axis=1)
        s1 = jnp.concatenate([s1_ref[kq] for kq in range(KQ)], axis=1)
        c0 = (c0_ref[0, :, 0] + c0_ref[1, :, 0]).astype(jnp.float32)
        c1 = (c1_ref[0, :, 0] + c1_ref[1, :, 0]).astype(jnp.float32)
        s01 = jnp.concatenate([s0, s1], axis=1)
        summ = jnp.dot(s01, w2_ref[...], preferred_element_type=jnp.float32)
        summ = summ + c0[:, None] * b2a_ref[...] + c1[:, None] * b2b_ref[...]
        z = jnp.maximum(
            jnp.dot(summ, u1a_ref[...], preferred_element_type=jnp.float32)
            + jnp.dot(ns_ref[...], u1b_ref[...],
                      preferred_element_type=jnp.float32)
            + u1_ref[...], 0.0)
        out_ref[...] = (jnp.dot(z, u2t_ref[...],
                                preferred_element_type=jnp.float32)
                        + u2_ref[...])

    return pl.pallas_call(
        body,
        grid=(nb,),
        in_specs=[
            pl.BlockSpec((KQ, bn, WQ), lambda i: (0, i, 0)),
            pl.BlockSpec((KQ, bn, WQ), lambda i: (0, i + nb, 0)),
            pl.BlockSpec((NC, bn, 16), lambda i: (0, i, 0)),
            pl.BlockSpec((NC, bn, 16), lambda i: (0, i + nb, 0)),
            pl.BlockSpec((bn, h), lambda i: (i, 0)),
            pl.BlockSpec((4 * h, h), lambda i: (0, 0)),
            pl.BlockSpec((1, h), lambda i: (0, 0)),
            pl.BlockSpec((1, h), lambda i: (0, 0)),
            pl.BlockSpec((h, 2 * h), lambda i: (0, 0)),
            pl.BlockSpec((h, 2 * h), lambda i: (0, 0)),
            pl.BlockSpec((1, 2 * h), lambda i: (0, 0)),
            pl.BlockSpec((2 * h, h), lambda i: (0, 0)),
            pl.BlockSpec((1, h), lambda i: (0, 0)),
        ],
        out_specs=pl.BlockSpec((bn, h), lambda i: (i, 0)),
        out_shape=jax.ShapeDtypeStruct((n, h), jnp.float32),
    )(sf, sf, cp, cp, node_states, W2ab, b2a, b2b, U1a, U1b, u1r, U2t, u2r)


def kernel(node_states, relations, W1, b1, W2, b2, U1, u1, U2, u2):
    n, h = node_states.shape
    nchunks = relations.shape[0] // CI
    rel2 = relations.reshape(nchunks, CI)

    A = W1[:, :h].T
    B = W1[:, h:].T
    Wcat = jnp.concatenate([A, B], axis=1)
    bcat = jnp.concatenate([b1, jnp.zeros_like(b1)]).reshape(1, 4 * h)
    W2ab = jnp.concatenate([W2[:h].T, W2[h:].T], axis=0)
    b2a = b2[:h].reshape(1, h)
    b2b = b2[h:].reshape(1, h)
    U1a = U1[:, :h].T
    U1b = U1[:, h:].T
    u1r = u1.reshape(1, 2 * h)
    U2t = U2.T
    u2r = u2.reshape(1, h)

    zrows = ((2 * n) // NS) // 8 * 8
    zb = jnp.zeros((zrows, WQ), jnp.bfloat16)
    z16 = jnp.zeros((zrows, 16), jnp.bfloat16)
    one16 = jnp.ones((CI, 16), jnp.bfloat16)

    tabs = _tc_tables(node_states, Wcat, bcat)
    sp, cp = _sc_scatter(tabs, rel2, zb, z16, one16, n, nchunks)
    sf = _sc_combine(sp, n)
    return _tc_update(sf, cp, node_states, W2ab, b2a, b2b,
                      U1a, U1b, u1r, U2t, u2r)
